# baseline (speedup 1.0000x reference)
import functools

import jax
import jax.numpy as jnp
from jax import lax
from jax.experimental import pallas as pl
from jax.experimental.pallas import tpu as pltpu
from jax.experimental.pallas import tpu_sc as plsc

N_NODES = 10000
DEG = 32
D = 128
OUT_DIM = 128
N_CLASSES = 40

NC = 2
NS = 16
NW = NC * NS
BP = 10240
BPW = BP // NW
NB = 64
NBURST = BPW // NB
NSTREAM = BPW // 4
NRING = 4


def _sc_sage_body(inputs_hbm, adjp_hbm, feat_hbm, self_hbm, sum_hbm,
                  iv, av, avc, sv, nbuf, sumbuf, gsem, ssem, swsem, wsem):
    c = lax.axis_index("c")
    s = lax.axis_index("s")
    wid = s * NC + c
    base = wid * BPW

    pltpu.sync_copy(inputs_hbm.at[pl.ds(base, BPW)], iv)

    for k in range(4):
        pltpu.async_copy(feat_hbm.at[iv.at[pl.ds(k * NB, NB)]],
                         sv.at[k], ssem)

    for k in range(2):
        pltpu.async_copy(adjp_hbm.at[iv.at[pl.ds(k * NB, NB)]],
                         av.at[k], gsem)
    for b2 in range(NBURST):
        hb = b2 % 2
        pltpu.make_async_copy(
            adjp_hbm.at[iv.at[pl.ds(0, NB)]], av.at[hb], gsem).wait()
        arow = b2 * (NB // 4)
        for i in range(NB):
            r, h = divmod(i, 4)
            avc[arow + r, pl.ds(32 * h, 16)] = av[hb, i, pl.ds(0, 16)]
            avc[arow + r, pl.ds(32 * h + 16, 16)] = av[hb, i, pl.ds(16, 16)]
        if b2 + 2 < NBURST:
            pltpu.async_copy(
                adjp_hbm.at[iv.at[pl.ds((b2 + 2) * NB, NB)]],
                av.at[hb], gsem)

    for t in range(NRING - 1):
        pltpu.async_copy(feat_hbm.at[avc.at[t]], nbuf.at[t], gsem)

    def quad(g, carry):
        for q in range(NRING):
            t = NRING * g + q
            pltpu.make_async_copy(
                feat_hbm.at[avc.at[0]], nbuf.at[q], gsem).wait()

            @pl.when(t + (NRING - 1) < NSTREAM)
            def _():
                pltpu.async_copy(
                    feat_hbm.at[avc.at[t + (NRING - 1)]],
                    nbuf.at[(q + NRING - 1) % NRING], gsem)

            lr4 = lax.rem(t, 8) * 4
            for nl in range(4):
                def red(k, accs):
                    return tuple(
                        accs[j] + nbuf[q, nl * 32 + k, pl.ds(j * 16, 16)]
                        for j in range(8)
                    )
                accs = tuple(jnp.zeros((16,), jnp.float32) for _ in range(8))
                accs = lax.fori_loop(0, 32, red, accs)
                for j in range(8):
                    sumbuf[lr4 + nl, pl.ds(j * 16, 16)] = accs[j]

            @pl.when(lax.rem(t, 8) == 7)
            def _():
                pltpu.async_copy(
                    sumbuf,
                    sum_hbm.at[pl.ds(pl.multiple_of(base + (t - 7) * 4, 32),
                                     32)],
                    wsem).wait()
        return carry

    lax.fori_loop(0, NSTREAM // NRING, quad, 0)

    for k in range(4):
        pltpu.make_async_copy(feat_hbm.at[iv.at[pl.ds(0, NB)]],
                              sv.at[k], ssem).wait()
    for k in range(4):
        pltpu.async_copy(sv.at[k], self_hbm.at[pl.ds(base + k * NB, NB)],
                         swsem)
    for k in range(4):
        pltpu.make_async_copy(sv.at[0], self_hbm.at[pl.ds(base, NB)],
                              swsem).wait()
    lastoff = 4 * NB
    pltpu.async_copy(feat_hbm.at[iv.at[pl.ds(lastoff, NB)]], sv.at[0], ssem)
    pltpu.make_async_copy(feat_hbm.at[iv.at[pl.ds(0, NB)]], sv.at[0],
                          ssem).wait()
    pltpu.async_copy(sv.at[0], self_hbm.at[pl.ds(base + lastoff, NB)], swsem)
    pltpu.make_async_copy(sv.at[0], self_hbm.at[pl.ds(base, NB)],
                          swsem).wait()


@functools.lru_cache(maxsize=1)
def _build_sc_kernel():
    mesh = plsc.VectorSubcoreMesh(core_axis_name="c", subcore_axis_name="s")
    return pl.kernel(
        _sc_sage_body,
        out_type=[
            jax.ShapeDtypeStruct((BP, D), jnp.float32),
            jax.ShapeDtypeStruct((BP, D), jnp.float32),
        ],
        mesh=mesh,
        scratch_types=[
            pltpu.VMEM((BPW,), jnp.int32),
            pltpu.VMEM((2, NB, 128), jnp.int32),
            pltpu.VMEM((NSTREAM, 128), jnp.int32),
            pltpu.VMEM((4, NB, D), jnp.float32),
            pltpu.VMEM((NRING, 128, D), jnp.float32),
            pltpu.VMEM((32, D), jnp.float32),
            pltpu.SemaphoreType.DMA,
            pltpu.SemaphoreType.DMA,
            pltpu.SemaphoreType.DMA,
            pltpu.SemaphoreType.DMA,
        ],
    )


_RB = 400


def _tc_body(self_ref, sum_ref, w1a_ref, w1s_ref, b1_ref, w3_ref, b3_ref, out_ref):
    x = jnp.dot(self_ref[...], w1a_ref[...], preferred_element_type=jnp.float32)
    x = x + jnp.dot(sum_ref[...], w1s_ref[...], preferred_element_type=jnp.float32)
    x = jnp.maximum(x + b1_ref[...], 0.0)
    l = jnp.dot(x, w3_ref[...], preferred_element_type=jnp.float32) + b3_ref[...]
    ss = jnp.sum(l * l, axis=1, keepdims=True)
    denom = jnp.maximum(jnp.sqrt(ss), 1e-12)
    out_ref[...] = l / denom


def kernel(inputs, adj, feat_data, W1, b1, W3, b3):
    B = inputs.shape[0]
    inputs_p = jnp.concatenate(
        [inputs.astype(jnp.int32), jnp.zeros((BP - B,), jnp.int32)])
    adj_p = jnp.pad(adj, ((0, 0), (0, 128 - DEG)))

    self_feat, sums = _build_sc_kernel()(inputs_p, adj_p, feat_data)

    w1a_t = W1[:, :D].T
    w1s_t = (W1[:, D:] * (1.0 / DEG)).T
    w3_t = W3.T
    b1_r = b1.reshape(1, OUT_DIM)
    b3_r = b3.reshape(1, N_CLASSES)

    logits = pl.pallas_call(
        _tc_body,
        out_shape=jax.ShapeDtypeStruct((B, N_CLASSES), jnp.float32),
        grid=(B // _RB,),
        in_specs=[
            pl.BlockSpec((_RB, D), lambda i: (i, 0)),
            pl.BlockSpec((_RB, D), lambda i: (i, 0)),
            pl.BlockSpec((D, OUT_DIM), lambda i: (0, 0)),
            pl.BlockSpec((D, OUT_DIM), lambda i: (0, 0)),
            pl.BlockSpec((1, OUT_DIM), lambda i: (0, 0)),
            pl.BlockSpec((OUT_DIM, N_CLASSES), lambda i: (0, 0)),
            pl.BlockSpec((1, N_CLASSES), lambda i: (0, 0)),
        ],
        out_specs=pl.BlockSpec((_RB, N_CLASSES), lambda i: (i, 0)),
    )(self_feat, sums, w1a_t, w1s_t, b1_r, w3_t, b3_r)

    return logits


# speedup vs baseline: 1.0272x; 1.0272x over previous
import functools

import jax
import jax.numpy as jnp
from jax import lax
from jax.experimental import pallas as pl
from jax.experimental.pallas import tpu as pltpu
from jax.experimental.pallas import tpu_sc as plsc

N_NODES = 10000
DEG = 32
D = 128
OUT_DIM = 128
N_CLASSES = 40

NC = 2
NS = 16
NW = NC * NS
BP = 10240
BPW = BP // NW
NB = 64
NBURST = BPW // NB
NSTREAM = BPW // 4
NRING = 4


def _sc_sage_body(inputs_hbm, adjp_hbm, feat_hbm, self_hbm, sum_hbm,
                  iv, av, avc, sv, nbuf, sumbuf, gsem, ssem, swsem, wsem):
    c = lax.axis_index("c")
    s = lax.axis_index("s")
    wid = s * NC + c
    base = wid * BPW

    pltpu.sync_copy(inputs_hbm.at[pl.ds(base, BPW)], iv)

    for k in range(4):
        pltpu.async_copy(feat_hbm.at[iv.at[pl.ds(k * NB, NB)]],
                         sv.at[k], ssem)

    for k in range(2):
        pltpu.async_copy(adjp_hbm.at[iv.at[pl.ds(k * NB, NB)]],
                         av.at[k], gsem)
    for b2 in range(NBURST):
        hb = b2 % 2
        pltpu.make_async_copy(
            adjp_hbm.at[iv.at[pl.ds(0, NB)]], av.at[hb], gsem).wait()
        arow = b2 * (NB // 4)
        for i in range(NB):
            r, h = divmod(i, 4)
            avc[arow + r, pl.ds(32 * h, 16)] = av[hb, i, pl.ds(0, 16)]
            avc[arow + r, pl.ds(32 * h + 16, 16)] = av[hb, i, pl.ds(16, 16)]
        if b2 + 2 < NBURST:
            pltpu.async_copy(
                adjp_hbm.at[iv.at[pl.ds((b2 + 2) * NB, NB)]],
                av.at[hb], gsem)

    for t in range(NRING - 1):
        pltpu.async_copy(feat_hbm.at[avc.at[t]], nbuf.at[t], gsem)

    def quad(g, carry):
        for q in range(NRING):
            t = NRING * g + q
            pltpu.make_async_copy(
                feat_hbm.at[avc.at[0]], nbuf.at[q], gsem).wait()

            @pl.when(t + (NRING - 1) < NSTREAM)
            def _():
                pltpu.async_copy(
                    feat_hbm.at[avc.at[t + (NRING - 1)]],
                    nbuf.at[(q + NRING - 1) % NRING], gsem)

            lr4 = lax.rem(t, 8) * 4
            for nl in range(4):
                def red(k, accs):
                    return tuple(
                        accs[j] + nbuf[q, nl * 32 + k, pl.ds(j * 16, 16)]
                        for j in range(8)
                    )
                accs = tuple(jnp.zeros((16,), jnp.float32) for _ in range(8))
                accs = lax.fori_loop(0, 32, red, accs)
                for j in range(8):
                    sumbuf[lr4 + nl, pl.ds(j * 16, 16)] = accs[j]

            @pl.when(lax.rem(t, 8) == 7)
            def _():
                pltpu.async_copy(
                    sumbuf,
                    sum_hbm.at[pl.ds(pl.multiple_of(base + (t - 7) * 4, 32),
                                     32)],
                    wsem).wait()
        return carry

    lax.fori_loop(0, NSTREAM // NRING, quad, 0)

    for k in range(4):
        pltpu.make_async_copy(feat_hbm.at[iv.at[pl.ds(0, NB)]],
                              sv.at[k], ssem).wait()
    for k in range(4):
        pltpu.async_copy(sv.at[k], self_hbm.at[pl.ds(base + k * NB, NB)],
                         swsem)
    for k in range(4):
        pltpu.make_async_copy(sv.at[0], self_hbm.at[pl.ds(base, NB)],
                              swsem).wait()
    lastoff = 4 * NB
    pltpu.async_copy(feat_hbm.at[iv.at[pl.ds(lastoff, NB)]], sv.at[0], ssem)
    pltpu.make_async_copy(feat_hbm.at[iv.at[pl.ds(0, NB)]], sv.at[0],
                          ssem).wait()
    pltpu.async_copy(sv.at[0], self_hbm.at[pl.ds(base + lastoff, NB)], swsem)
    pltpu.make_async_copy(sv.at[0], self_hbm.at[pl.ds(base, NB)],
                          swsem).wait()


@functools.lru_cache(maxsize=1)
def _build_sc_kernel():
    mesh = plsc.VectorSubcoreMesh(core_axis_name="c", subcore_axis_name="s")
    return pl.kernel(
        _sc_sage_body,
        out_type=[
            jax.ShapeDtypeStruct((BP, D), jnp.float32),
            jax.ShapeDtypeStruct((BP, D), jnp.float32),
        ],
        mesh=mesh,
        scratch_types=[
            pltpu.VMEM((BPW,), jnp.int32),
            pltpu.VMEM((2, NB, 128), jnp.int32),
            pltpu.VMEM((NSTREAM, 128), jnp.int32),
            pltpu.VMEM((4, NB, D), jnp.float32),
            pltpu.VMEM((NRING, 128, D), jnp.float32),
            pltpu.VMEM((32, D), jnp.float32),
            pltpu.SemaphoreType.DMA,
            pltpu.SemaphoreType.DMA,
            pltpu.SemaphoreType.DMA,
            pltpu.SemaphoreType.DMA,
        ],
    )


_RB = 512


def _tc_body(self_ref, sum_ref, w1a_ref, w1s_ref, b1_ref, w3_ref, b3_ref, out_ref):
    x = jnp.dot(self_ref[...], w1a_ref[...], preferred_element_type=jnp.float32)
    x = x + jnp.dot(sum_ref[...], w1s_ref[...], preferred_element_type=jnp.float32)
    x = jnp.maximum(x + b1_ref[...], 0.0)
    l = jnp.dot(x, w3_ref[...], preferred_element_type=jnp.float32) + b3_ref[...]
    ss = jnp.sum(l * l, axis=1, keepdims=True)
    denom = jnp.maximum(jnp.sqrt(ss), 1e-12)
    out_ref[...] = l / denom


def kernel(inputs, adj, feat_data, W1, b1, W3, b3):
    B = inputs.shape[0]
    inputs_p = jnp.concatenate(
        [inputs.astype(jnp.int32), jnp.zeros((BP - B,), jnp.int32)])
    adj_p = jnp.pad(adj, ((0, 0), (0, 128 - DEG)))

    self_feat, sums = _build_sc_kernel()(inputs_p, adj_p, feat_data)

    w1a_t = W1[:, :D].T
    w1s_t = (W1[:, D:] * (1.0 / DEG)).T
    w3_t = jnp.pad(W3.T, ((0, 0), (0, 128 - N_CLASSES)))
    b1_r = b1.reshape(1, OUT_DIM)
    b3_r = jnp.pad(b3, (0, 128 - N_CLASSES)).reshape(1, 128)

    logits = pl.pallas_call(
        _tc_body,
        out_shape=jax.ShapeDtypeStruct((BP, 128), jnp.float32),
        grid=(BP // _RB,),
        in_specs=[
            pl.BlockSpec((_RB, D), lambda i: (i, 0)),
            pl.BlockSpec((_RB, D), lambda i: (i, 0)),
            pl.BlockSpec((D, OUT_DIM), lambda i: (0, 0)),
            pl.BlockSpec((D, OUT_DIM), lambda i: (0, 0)),
            pl.BlockSpec((1, OUT_DIM), lambda i: (0, 0)),
            pl.BlockSpec((OUT_DIM, 128), lambda i: (0, 0)),
            pl.BlockSpec((1, 128), lambda i: (0, 0)),
        ],
        out_specs=pl.BlockSpec((_RB, 128), lambda i: (i, 0)),
    )(self_feat, sums, w1a_t, w1s_t, b1_r, w3_t, b3_r)

    return logits[:B, :N_CLASSES]
